# Initial kernel scaffold; baseline (speedup 1.0000x reference)
import jax
import jax.numpy as jnp
from jax.experimental import pallas as pl


def kernel(inputs, user_emb, user_bias, movie_emb, movie_bias):
    raise NotImplementedError("write your pallas kernel here")


# speedup vs baseline: 1.0964x; 1.0964x over previous
import functools

import jax
import jax.numpy as jnp
from jax import lax
from jax.experimental import pallas as pl
from jax.experimental.pallas import tpu as pltpu
from jax.experimental.pallas import tpu_sc as plsc

NC = 2
NS = 16
L = 16
NW = NC * NS

B = 16384
D = 128
G = 128
PER_W = B // NW
NCHUNK = PER_W // G


def _body(uidx_hbm, midx_hbm, uemb_hbm, memb_hbm, ubias_hbm, mbias_hbm,
          out_hbm,
          uidx_v, midx_v, urows_v, mrows_v, ubias_v, mbias_v, out_v,
          sem_u, sem_m, sem_ub, sem_mb):
  wid = lax.axis_index("s") * NC + lax.axis_index("c")
  base = wid * PER_W

  iot = lax.iota(jnp.int32, L)

  def chunk_body(j, carry):
    off = base + j * G
    pltpu.sync_copy(uidx_hbm.at[pl.ds(off, G)], uidx_v)
    pltpu.sync_copy(midx_hbm.at[pl.ds(off, G)], midx_v)
    cp_u = pltpu.async_copy(uemb_hbm.at[uidx_v], urows_v, sem_u)
    cp_m = pltpu.async_copy(memb_hbm.at[midx_v], mrows_v, sem_m)
    cp_ub = pltpu.async_copy(ubias_hbm.at[uidx_v], ubias_v, sem_ub)
    cp_mb = pltpu.async_copy(mbias_hbm.at[midx_v], mbias_v, sem_mb)
    cp_u.wait()
    cp_m.wait()
    cp_ub.wait()
    cp_mb.wait()

    for g in range(G // L):
      tot = jnp.zeros((L,), jnp.float32)
      for i in range(L):
        row = g * L + i
        acc = urows_v[row, pl.ds(0, L)] * mrows_v[row, pl.ds(0, L)]
        for k in range(1, D // L):
          acc = acc + urows_v[row, pl.ds(k * L, L)] * mrows_v[row, pl.ds(k * L, L)]
        s = jnp.sum(acc)
        tot = jnp.where(iot == i, s, tot)
      x = tot + ubias_v[pl.ds(g * L, L)] + mbias_v[pl.ds(g * L, L)]
      y = 5.0 / (1.0 + jnp.exp(-x))
      out_v[pl.ds(j * G + g * L, L)] = y
    return carry

  lax.fori_loop(0, NCHUNK, chunk_body, 0, unroll=False)
  pltpu.sync_copy(out_v, out_hbm.at[pl.ds(base, PER_W)])


@functools.partial(jax.jit, donate_argnums=())
def _run(uidx, midx, uemb, memb, ubias, mbias):
  mesh = plsc.VectorSubcoreMesh(core_axis_name="c", subcore_axis_name="s",
                                num_cores=NC, num_subcores=NS)
  fn = pl.kernel(
      _body,
      out_type=jax.ShapeDtypeStruct((B,), jnp.float32),
      mesh=mesh,
      compiler_params=pltpu.CompilerParams(needs_layout_passes=False),
      scratch_types=[
          pltpu.VMEM((G,), jnp.int32),
          pltpu.VMEM((G,), jnp.int32),
          pltpu.VMEM((G, D), jnp.float32),
          pltpu.VMEM((G, D), jnp.float32),
          pltpu.VMEM((G,), jnp.float32),
          pltpu.VMEM((G,), jnp.float32),
          pltpu.VMEM((PER_W,), jnp.float32),
          pltpu.SemaphoreType.DMA,
          pltpu.SemaphoreType.DMA,
          pltpu.SemaphoreType.DMA,
          pltpu.SemaphoreType.DMA,
      ],
  )
  return fn(uidx, midx, uemb, memb, ubias, mbias)


def kernel(inputs, user_emb, user_bias, movie_emb, movie_bias):
  uidx = inputs[:, 0].astype(jnp.int32)
  midx = inputs[:, 1].astype(jnp.int32)
  out = _run(uidx, midx, user_emb, movie_emb,
             user_bias.reshape(-1), movie_bias.reshape(-1))
  return out.reshape(B, 1)


# speedup vs baseline: 1.2551x; 1.1448x over previous
import functools

import jax
import jax.numpy as jnp
from jax import lax
from jax.experimental import pallas as pl
from jax.experimental.pallas import tpu as pltpu
from jax.experimental.pallas import tpu_sc as plsc

NC = 2
NS = 16
L = 16
NW = NC * NS

B = 16384
D = 128
G = 64
PER_W = B // NW
NCHUNK = PER_W // G


def _body(uidx_hbm, midx_hbm, uemb_hbm, memb_hbm, ubias_hbm, mbias_hbm,
          out_hbm,
          uidx_v, midx_v, urows_v, mrows_v, ubias_v, mbias_v, out_v,
          sem0, sem1):
  wid = lax.axis_index("s") * NC + lax.axis_index("c")
  base = wid * PER_W

  iot = lax.iota(jnp.int32, L)
  sems = (sem0, sem1)

  def launch(j, b):
    sem = sems[b]
    pltpu.async_copy(uemb_hbm.at[uidx_v.at[j]], urows_v.at[b], sem)
    pltpu.async_copy(memb_hbm.at[midx_v.at[j]], mrows_v.at[b], sem)
    pltpu.async_copy(ubias_hbm.at[uidx_v.at[j]], ubias_v.at[b], sem)
    pltpu.async_copy(mbias_hbm.at[midx_v.at[j]], mbias_v.at[b], sem)

  def drain(j, b):
    sem = sems[b]
    pltpu.make_async_copy(uemb_hbm.at[uidx_v.at[j]], urows_v.at[b], sem).wait()
    pltpu.make_async_copy(memb_hbm.at[midx_v.at[j]], mrows_v.at[b], sem).wait()
    pltpu.make_async_copy(ubias_hbm.at[uidx_v.at[j]], ubias_v.at[b], sem).wait()
    pltpu.make_async_copy(mbias_hbm.at[midx_v.at[j]], mbias_v.at[b], sem).wait()

  def compute(j, b):
    for g in range(G // L):
      tot = jnp.zeros((L,), jnp.float32)
      for i in range(L):
        row = g * L + i
        acc = urows_v[b, row, pl.ds(0, L)] * mrows_v[b, row, pl.ds(0, L)]
        for k in range(1, D // L):
          acc = acc + urows_v[b, row, pl.ds(k * L, L)] * mrows_v[b, row, pl.ds(k * L, L)]
        s = jnp.sum(acc)
        tot = jnp.where(iot == i, s, tot)
      x = tot + ubias_v[b, pl.ds(g * L, L)] + mbias_v[b, pl.ds(g * L, L)]
      y = 5.0 / (1.0 + jnp.exp(-x))
      out_v[pl.ds(j * G + g * L, L)] = y

  pltpu.sync_copy(uidx_hbm.at[wid], uidx_v)
  pltpu.sync_copy(midx_hbm.at[wid], midx_v)

  launch(0, 0)

  def pair_body(t, carry):
    j0 = 2 * t
    j1 = j0 + 1
    launch(j1, 1)
    drain(j0, 0)
    compute(j0, 0)

    @pl.when(j1 + 1 < NCHUNK)
    def _():
      launch(j1 + 1, 0)

    drain(j1, 1)
    compute(j1, 1)
    return carry

  lax.fori_loop(0, NCHUNK // 2, pair_body, 0, unroll=False)
  pltpu.sync_copy(out_v, out_hbm.at[pl.ds(base, PER_W)])


@functools.partial(jax.jit, donate_argnums=())
def _run(uidx, midx, uemb, memb, ubias, mbias):
  mesh = plsc.VectorSubcoreMesh(core_axis_name="c", subcore_axis_name="s",
                                num_cores=NC, num_subcores=NS)
  fn = pl.kernel(
      _body,
      out_type=jax.ShapeDtypeStruct((B,), jnp.float32),
      mesh=mesh,
      compiler_params=pltpu.CompilerParams(needs_layout_passes=False),
      scratch_types=[
          pltpu.VMEM((NCHUNK, G), jnp.int32),
          pltpu.VMEM((NCHUNK, G), jnp.int32),
          pltpu.VMEM((2, G, D), jnp.float32),
          pltpu.VMEM((2, G, D), jnp.float32),
          pltpu.VMEM((2, G), jnp.float32),
          pltpu.VMEM((2, G), jnp.float32),
          pltpu.VMEM((PER_W,), jnp.float32),
          pltpu.SemaphoreType.DMA,
          pltpu.SemaphoreType.DMA,
      ],
  )
  return fn(uidx, midx, uemb, memb, ubias, mbias)


def kernel(inputs, user_emb, user_bias, movie_emb, movie_bias):
  uidx = inputs[:, 0].astype(jnp.int32).reshape(NW, NCHUNK, G)
  midx = inputs[:, 1].astype(jnp.int32).reshape(NW, NCHUNK, G)
  out = _run(uidx, midx, user_emb, movie_emb,
             user_bias.reshape(-1), movie_bias.reshape(-1))
  return out.reshape(B, 1)
